# baseline (speedup 1.0000x reference)
import jax
import jax.numpy as jnp
from jax import lax
from jax.experimental import pallas as pl
from jax.experimental.pallas import tpu as pltpu
from jax.experimental.pallas import tpu_sc as plsc

D = 64
SCALE = 8.0
NC = 2
NS = 16
NW = NC * NS
L = 16

S = 4096
T = 200
SBLK = 256
UNITS = T * (S // SBLK)
U_PER_W = UNITS // NW
N_PAIRS = U_PER_W // 2
SB_PER_T = S // SBLK


def _emb_body(xt_hbm, lut2_hbm, out_hbm,
              xblk0, xblk1, idx0, idx1, cv0, cv1,
              rin0, rin1, rout0, rout1,
              gsem0, gsem1, osem0, osem1):
    wid = lax.axis_index("s") * NC + lax.axis_index("c")
    u_base = wid * U_PER_W
    xblk = (xblk0, xblk1)
    idx = (idx0, idx1)
    cvec = (cv0, cv1)
    rin = (rin0, rin1)
    rout = (rout0, rout1)
    gsem = (gsem0, gsem1)
    osem = (osem0, osem1)

    def start_gather(u, b):
        t = u // SB_PER_T
        s0 = (u % SB_PER_T) * SBLK
        t8 = (t // 8) * 8
        tq = t - t8
        pltpu.sync_copy(xt_hbm.at[pl.ds(t8, 8), pl.ds(s0, SBLK)], xblk[b])

        def prep(k, c):
            sl = pl.ds(k * L, L)
            xb = xblk[b][tq, sl]
            idx[b][sl] = lax.shift_right_logical(xb, 1)
            cvec[b][sl] = (xb & 1) * D
            return c

        lax.fori_loop(0, SBLK // L, prep, 0)
        pltpu.async_copy(lut2_hbm.at[idx[b]], rin[b], gsem[b])

    def wait_gather(b):
        pltpu.make_async_copy(lut2_hbm.at[idx[b]], rin[b], gsem[b]).wait()

    def start_out(u, b):
        t = u // SB_PER_T
        s0 = (u % SB_PER_T) * SBLK
        pltpu.async_copy(rout[b], out_hbm.at[t, slice(None), pl.ds(s0, SBLK)],
                         osem[b])

    def wait_out(b):
        pltpu.make_async_copy(
            rout[b], out_hbm.at[0, slice(None), pl.ds(0, SBLK)],
            osem[b]).wait()

    def transpose_scale(b):
        def kloop(k, c):
            sl = pl.ds(k * L, L)
            jv = lax.iota(jnp.int32, L) + k * L
            cv = cvec[b][sl]
            for d in range(D):
                val = plsc.load_gather(rin[b], [jv, cv + d])
                rout[b][d, sl] = val * SCALE
            return c

        lax.fori_loop(0, SBLK // L, kloop, 0)

    def phase(u, b, first, last):
        if not last:
            start_gather(u + 1, 1 - b)
        wait_gather(b)
        if not first:
            wait_out(b)
        transpose_scale(b)
        start_out(u, b)

    start_gather(u_base, 0)
    phase(u_base, 0, True, False)
    phase(u_base + 1, 1, True, False)

    def pair(i, c):
        u = u_base + i * 2
        phase(u, 0, False, False)
        phase(u + 1, 1, False, False)
        return c

    lax.fori_loop(1, N_PAIRS - 1, pair, 0)
    phase(u_base + U_PER_W - 2, 0, False, False)
    phase(u_base + U_PER_W - 1, 1, False, True)
    wait_out(0)
    wait_out(1)


@jax.jit
def kernel(x, lut):
    xt = jnp.transpose(x)
    lut2 = jnp.reshape(lut, (500000, 128))
    call = pl.kernel(
        _emb_body,
        out_type=jax.ShapeDtypeStruct((T, D, S), jnp.float32),
        mesh=plsc.VectorSubcoreMesh(core_axis_name="c", subcore_axis_name="s"),
        scratch_types=[
            pltpu.VMEM((8, SBLK), jnp.int32),
            pltpu.VMEM((8, SBLK), jnp.int32),
            pltpu.VMEM((SBLK,), jnp.int32),
            pltpu.VMEM((SBLK,), jnp.int32),
            pltpu.VMEM((SBLK,), jnp.int32),
            pltpu.VMEM((SBLK,), jnp.int32),
            pltpu.VMEM((SBLK, 128), jnp.float32),
            pltpu.VMEM((SBLK, 128), jnp.float32),
            pltpu.VMEM((D, SBLK), jnp.float32),
            pltpu.VMEM((D, SBLK), jnp.float32),
            pltpu.SemaphoreType.DMA,
            pltpu.SemaphoreType.DMA,
            pltpu.SemaphoreType.DMA,
            pltpu.SemaphoreType.DMA,
        ],
        compiler_params=pltpu.CompilerParams(
            use_tc_tiling_on_sc=True, needs_layout_passes=False),
    )
    out3 = call(xt, lut2)
    return jnp.transpose(out3, (2, 0, 1))


# speedup vs baseline: 1.4151x; 1.4151x over previous
import jax
import jax.numpy as jnp
from jax import lax
from jax.experimental import pallas as pl
from jax.experimental.pallas import tpu as pltpu
from jax.experimental.pallas import tpu_sc as plsc

D = 64
SCALE = 8.0
NC = 2
NS = 16
NW = NC * NS
L = 16

S = 4096
T = 200
SBLK = 256
UNITS = T * (S // SBLK)
U_PER_W = UNITS // NW
N_PAIRS = U_PER_W // 2
SB_PER_T = S // SBLK


def _emb_body(xt_hbm, lut2_hbm, out_hbm,
              xblk0, xblk1, idx0, idx1, cv0, cv1,
              rin0, rin1, rout0, rout1,
              gsem0, gsem1, osem0, osem1):
    wid = lax.axis_index("s") * NC + lax.axis_index("c")
    u_base = wid * U_PER_W
    xblk = (xblk0, xblk1)
    idx = (idx0, idx1)
    cvec = (cv0, cv1)
    rin = (rin0, rin1)
    rout = (rout0, rout1)
    gsem = (gsem0, gsem1)
    osem = (osem0, osem1)

    def start_gather(u, b):
        t = u // SB_PER_T
        s0 = (u % SB_PER_T) * SBLK
        t8 = (t // 8) * 8
        tq = t - t8
        pltpu.sync_copy(xt_hbm.at[pl.ds(t8, 8), pl.ds(s0, SBLK)], xblk[b])

        def prep(k, c):
            sl = pl.ds(k * L, L)
            xb = xblk[b][tq, sl]
            idx[b][sl] = lax.shift_right_logical(xb, 1)
            cvec[b][sl] = (xb & 1) * D
            return c

        lax.fori_loop(0, SBLK // L, prep, 0)
        pltpu.async_copy(lut2_hbm.at[idx[b]], rin[b], gsem[b])

    def wait_gather(b):
        pltpu.make_async_copy(lut2_hbm.at[idx[b]], rin[b], gsem[b]).wait()

    def start_out(u, b):
        t = u // SB_PER_T
        s0 = (u % SB_PER_T) * SBLK
        pltpu.async_copy(rout[b], out_hbm.at[t, slice(None), pl.ds(s0, SBLK)],
                         osem[b])

    def wait_out(b):
        pltpu.make_async_copy(
            rout[b], out_hbm.at[0, slice(None), pl.ds(0, SBLK)],
            osem[b]).wait()

    def transpose_scale(b):
        def kloop(k, c):
            sl = pl.ds(k * L, L)
            jv = lax.iota(jnp.int32, L) + k * L
            cv = cvec[b][sl]
            G = 8
            for g in range(D // G):
                vals = [plsc.load_gather(rin[b], [jv, cv + (g * G + dd)])
                        for dd in range(G)]
                for dd in range(G):
                    rout[b][g * G + dd, sl] = vals[dd] * SCALE
            return c

        lax.fori_loop(0, SBLK // L, kloop, 0)

    def phase(u, b, first, last):
        if not last:
            start_gather(u + 1, 1 - b)
        wait_gather(b)
        if not first:
            wait_out(b)
        transpose_scale(b)
        start_out(u, b)

    start_gather(u_base, 0)
    phase(u_base, 0, True, False)
    phase(u_base + 1, 1, True, False)

    def pair(i, c):
        u = u_base + i * 2
        phase(u, 0, False, False)
        phase(u + 1, 1, False, False)
        return c

    lax.fori_loop(1, N_PAIRS - 1, pair, 0)
    phase(u_base + U_PER_W - 2, 0, False, False)
    phase(u_base + U_PER_W - 1, 1, False, True)
    wait_out(0)
    wait_out(1)


@jax.jit
def kernel(x, lut):
    xt = jnp.transpose(x)
    lut2 = jnp.reshape(lut, (500000, 128))
    call = pl.kernel(
        _emb_body,
        out_type=jax.ShapeDtypeStruct((T, D, S), jnp.float32),
        mesh=plsc.VectorSubcoreMesh(core_axis_name="c", subcore_axis_name="s"),
        scratch_types=[
            pltpu.VMEM((8, SBLK), jnp.int32),
            pltpu.VMEM((8, SBLK), jnp.int32),
            pltpu.VMEM((SBLK,), jnp.int32),
            pltpu.VMEM((SBLK,), jnp.int32),
            pltpu.VMEM((SBLK,), jnp.int32),
            pltpu.VMEM((SBLK,), jnp.int32),
            pltpu.VMEM((SBLK, 128), jnp.float32),
            pltpu.VMEM((SBLK, 128), jnp.float32),
            pltpu.VMEM((D, SBLK), jnp.float32),
            pltpu.VMEM((D, SBLK), jnp.float32),
            pltpu.SemaphoreType.DMA,
            pltpu.SemaphoreType.DMA,
            pltpu.SemaphoreType.DMA,
            pltpu.SemaphoreType.DMA,
        ],
        compiler_params=pltpu.CompilerParams(
            use_tc_tiling_on_sc=True, needs_layout_passes=False),
    )
    out3 = call(xt, lut2)
    return jnp.transpose(out3, (2, 0, 1))


# speedup vs baseline: 1.5387x; 1.0873x over previous
import jax
import jax.numpy as jnp
from jax import lax
from jax.experimental import pallas as pl
from jax.experimental.pallas import tpu as pltpu
from jax.experimental.pallas import tpu_sc as plsc

D = 64
SCALE = 8.0
NC = 2
NS = 16
NW = NC * NS
L = 16

S = 4096
T = 200
SBLK = 256
UNITS = T * (S // SBLK)
U_PER_W = UNITS // NW
N_PAIRS = U_PER_W // 2
SB_PER_T = S // SBLK


def _emb_body(xt_hbm, lut2_hbm, out_hbm,
              xblk0, xblk1, idx0, idx1, cv0, cv1,
              rin0, rin1, rout0, rout1,
              gsem0, gsem1, osem0, osem1):
    wid = lax.axis_index("s") * NC + lax.axis_index("c")
    u_base = wid * U_PER_W
    xblk = (xblk0, xblk1)
    idx = (idx0, idx1)
    cvec = (cv0, cv1)
    rin = (rin0, rin1)
    rout = (rout0, rout1)
    gsem = (gsem0, gsem1)
    osem = (osem0, osem1)

    def start_gather(u, b):
        t = u // SB_PER_T
        s0 = (u % SB_PER_T) * SBLK
        t8 = (t // 8) * 8
        tq = t - t8
        pltpu.sync_copy(xt_hbm.at[pl.ds(t8, 8), pl.ds(s0, SBLK)], xblk[b])

        def prep(k, c):
            sl = pl.ds(k * L, L)
            xb = xblk[b][tq, sl]
            idx[b][sl] = lax.shift_right_logical(xb, 1)
            cvec[b][sl] = (xb & 1) * D
            return c

        lax.fori_loop(0, SBLK // L, prep, 0)
        pltpu.async_copy(lut2_hbm.at[idx[b]], rin[b], gsem[b])

    def wait_gather(b):
        pltpu.make_async_copy(lut2_hbm.at[idx[b]], rin[b], gsem[b]).wait()

    def start_out(u, b):
        t = u // SB_PER_T
        s0 = (u % SB_PER_T) * SBLK
        pltpu.async_copy(rout[b], out_hbm.at[t, slice(None), pl.ds(s0, SBLK)],
                         osem[b])

    def wait_out(b):
        pltpu.make_async_copy(
            rout[b], out_hbm.at[0, slice(None), pl.ds(0, SBLK)],
            osem[b]).wait()

    def transpose_scale(b):
        iot = lax.iota(jnp.int32, L)
        rots = [(iot + k2) & (L - 1) for k2 in range(L)]

        def kloop(k, c):
            sl = pl.ds(k * L, L)
            jv = iot + k * L
            cv = cvec[b][sl]
            sv = iot + k * L
            for db in range(0, D, L):
                for k2 in range(L):
                    colv = cv + (rots[k2] + db)
                    val = plsc.load_gather(rin[b], [jv, colv])
                    plsc.store_scatter(rout[b], [rots[k2] + db, sv],
                                       val * SCALE)
            return c

        lax.fori_loop(0, SBLK // L, kloop, 0)

    def phase(u, b, first, last):
        if not last:
            start_gather(u + 1, 1 - b)
        wait_gather(b)
        if not first:
            wait_out(b)
        transpose_scale(b)
        start_out(u, b)

    start_gather(u_base, 0)
    phase(u_base, 0, True, False)
    phase(u_base + 1, 1, True, False)

    def pair(i, c):
        u = u_base + i * 2
        phase(u, 0, False, False)
        phase(u + 1, 1, False, False)
        return c

    lax.fori_loop(1, N_PAIRS - 1, pair, 0)
    phase(u_base + U_PER_W - 2, 0, False, False)
    phase(u_base + U_PER_W - 1, 1, False, True)
    wait_out(0)
    wait_out(1)


@jax.jit
def kernel(x, lut):
    xt = jnp.transpose(x)
    lut2 = jnp.reshape(lut, (500000, 128))
    call = pl.kernel(
        _emb_body,
        out_type=jax.ShapeDtypeStruct((T, D, S), jnp.float32),
        mesh=plsc.VectorSubcoreMesh(core_axis_name="c", subcore_axis_name="s"),
        scratch_types=[
            pltpu.VMEM((8, SBLK), jnp.int32),
            pltpu.VMEM((8, SBLK), jnp.int32),
            pltpu.VMEM((SBLK,), jnp.int32),
            pltpu.VMEM((SBLK,), jnp.int32),
            pltpu.VMEM((SBLK,), jnp.int32),
            pltpu.VMEM((SBLK,), jnp.int32),
            pltpu.VMEM((SBLK, 128), jnp.float32),
            pltpu.VMEM((SBLK, 128), jnp.float32),
            pltpu.VMEM((D, SBLK), jnp.float32),
            pltpu.VMEM((D, SBLK), jnp.float32),
            pltpu.SemaphoreType.DMA,
            pltpu.SemaphoreType.DMA,
            pltpu.SemaphoreType.DMA,
            pltpu.SemaphoreType.DMA,
        ],
        compiler_params=pltpu.CompilerParams(
            use_tc_tiling_on_sc=True, needs_layout_passes=False),
    )
    out3 = call(xt, lut2)
    return jnp.transpose(out3, (2, 0, 1))


# speedup vs baseline: 1.6450x; 1.0691x over previous
import jax
import jax.numpy as jnp
from jax import lax
from jax.experimental import pallas as pl
from jax.experimental.pallas import tpu as pltpu
from jax.experimental.pallas import tpu_sc as plsc

D = 64
SCALE = 8.0
NC = 2
NS = 16
NW = NC * NS
L = 16

B_TOTAL = 4096 * 200
B_PER_W = B_TOTAL // NW
CHUNK = 400
N_CHUNKS = B_PER_W // CHUNK
N_PAIRS = N_CHUNKS // 2
ROWS_PER_IT = 8
ROW_VECS = D // L


def _emb_body(x_hbm, lut_hbm, out_hbm,
              idx0, idx1, rin0, rin1, rout0, rout1,
              gsem0, gsem1, osem0, osem1):
    wid = lax.axis_index("s") * NC + lax.axis_index("c")
    base = wid * B_PER_W
    idx = (idx0, idx1)
    rin = (rin0, rin1)
    rout = (rout0, rout1)
    gsem = (gsem0, gsem1)
    osem = (osem0, osem1)

    def start_gather(g, b):
        off = base + g * CHUNK
        pltpu.sync_copy(x_hbm.at[pl.ds(off, CHUNK)], idx[b])
        pltpu.async_copy(lut_hbm.at[idx[b]], rin[b], gsem[b])

    def wait_gather(b):
        pltpu.make_async_copy(lut_hbm.at[idx[b]], rin[b], gsem[b]).wait()

    def start_out(g, b):
        off = base + g * CHUNK
        pltpu.async_copy(rout[b], out_hbm.at[pl.ds(off, CHUNK)], osem[b])

    def wait_out(b):
        pltpu.make_async_copy(
            rout[b], out_hbm.at[pl.ds(base, CHUNK)], osem[b]).wait()

    def scale(b):
        def body(r, c):
            for u in range(ROWS_PER_IT):
                row = r * ROWS_PER_IT + u
                for j in range(ROW_VECS):
                    sl = pl.ds(j * L, L)
                    rout[b][row, sl] = rin[b][row, sl] * SCALE
            return c
        lax.fori_loop(0, CHUNK // ROWS_PER_IT, body, 0)

    def phase(g, b, first, last):
        if not last:
            start_gather(g + 1, 1 - b)
        wait_gather(b)
        if not first:
            wait_out(b)
        scale(b)
        start_out(g, b)

    start_gather(0, 0)
    phase(0, 0, True, False)
    phase(1, 1, True, False)

    def pair(i, c):
        g = i * 2
        phase(g, 0, False, False)
        phase(g + 1, 1, False, False)
        return c

    lax.fori_loop(1, N_PAIRS - 1, pair, 0)
    phase(N_CHUNKS - 2, 0, False, False)
    phase(N_CHUNKS - 1, 1, False, True)
    wait_out(0)
    wait_out(1)


@jax.jit
def kernel(x, lut):
    xf = x.reshape(-1).astype(jnp.int32)
    call = pl.kernel(
        _emb_body,
        out_type=jax.ShapeDtypeStruct((B_TOTAL, D), jnp.float32),
        mesh=plsc.VectorSubcoreMesh(core_axis_name="c", subcore_axis_name="s"),
        scratch_types=[
            pltpu.VMEM((CHUNK,), jnp.int32),
            pltpu.VMEM((CHUNK,), jnp.int32),
            pltpu.VMEM((CHUNK, D), jnp.float32),
            pltpu.VMEM((CHUNK, D), jnp.float32),
            pltpu.VMEM((CHUNK, D), jnp.float32),
            pltpu.VMEM((CHUNK, D), jnp.float32),
            pltpu.SemaphoreType.DMA,
            pltpu.SemaphoreType.DMA,
            pltpu.SemaphoreType.DMA,
            pltpu.SemaphoreType.DMA,
        ],
        compiler_params=pltpu.CompilerParams(use_tc_tiling_on_sc=False),
    )
    out = call(xf, lut)
    return out.reshape(x.shape[0], x.shape[1], D)
